# Initial kernel scaffold; baseline (speedup 1.0000x reference)
import jax
import jax.numpy as jnp
from jax.experimental import pallas as pl


def kernel(x, enc_up, enc_low, cb_up, cb_low, dec_up, dec_low):
    raise NotImplementedError("write your pallas kernel here")


# speedup vs baseline: 1.0185x; 1.0185x over previous
import numpy as np
import jax
import jax.numpy as jnp
from jax.experimental import pallas as pl
from jax.experimental.pallas import tpu as pltpu

_NB = 512
_CH = 256
_C = 512
_OUT = 263
_UP = 156

_perm = np.random.default_rng(42).permutation(_OUT)
_UPIDX = np.sort(_perm[:_UP])
_LOIDX = np.sort(_perm[_UP:])


def _mm(x, w):
    return jax.lax.dot_general(x, w, (((1,), (0,)), ((), ())),
                               preferred_element_type=jnp.float32)


def _conv3(h, w, b, dil):
    T, bB, C = h.shape
    N = w.shape[2]
    z = jnp.zeros((dil, bB, C), jnp.float32)
    hp = jnp.concatenate([z, h, z], axis=0)
    acc = _mm(hp[0:T].reshape(T * bB, C), w[0])
    acc = acc + _mm(hp[dil:dil + T].reshape(T * bB, C), w[1])
    acc = acc + _mm(hp[2 * dil:2 * dil + T].reshape(T * bB, C), w[2])
    return (acc + b).reshape(T, bB, N)


def _resblock(hx, w1, b1, w2, b2, dil):
    T, bB, C = hx.shape
    a = jnp.maximum(hx, 0.0)
    a = jnp.maximum(_conv3(a, w1, b1, dil), 0.0)
    h = _mm(a.reshape(T * bB, C), w2) + b2
    return hx + h.reshape(T, bB, C)


def _in_conv_body(x_ref, w_ref, b_ref, o_ref):
    o_ref[...] = jnp.maximum(_conv3(x_ref[...], w_ref[...], b_ref[...], 1), 0.0)


def _dec_stage_body(x_ref, *refs):
    o_ref = refs[-1]
    ws = [r[...] for r in refs[:-1]]
    h = x_ref[...]
    k = 0
    for dil in (9, 3, 1):
        h = _resblock(h, ws[k], ws[k + 1], ws[k + 2], ws[k + 3], dil)
        k += 4
    T, bB, C = h.shape
    h = jnp.broadcast_to(h[:, None], (T, 2, bB, C)).reshape(2 * T, bB, C)
    o_ref[...] = _conv3(h, ws[k], ws[k + 1], 1)


def _dec_out_body(x_ref, mw_ref, mb_ref, ow_ref, ob_ref, o_ref):
    h = jnp.maximum(_conv3(x_ref[...], mw_ref[...], mb_ref[...], 1), 0.0)
    o_ref[...] = _conv3(h, ow_ref[...], ob_ref[...], 1)


def _dec_out_add_body(x_ref, other_ref, mw_ref, mb_ref, ow_ref, ob_ref, o_ref):
    h = jnp.maximum(_conv3(x_ref[...], mw_ref[...], mb_ref[...], 1), 0.0)
    o_ref[...] = other_ref[...] + _conv3(h, ow_ref[...], ob_ref[...], 1)


def _quant_body(x_ref, cbt_ref, cb_ref, prev_ref, xd_ref, loss_ref, perp_ref):
    T, B, CH = x_ref.shape
    xf = x_ref[...].reshape(T * B, CH)
    cbt = cbt_ref[...]
    cb = cb_ref[...]
    mm = _mm(xf, cbt)
    xn = jnp.sum(xf * xf, axis=1, keepdims=True)
    cn = jnp.sum(cb * cb, axis=1)
    dist = xn - 2.0 * mm + cn[None, :]
    m = jnp.min(dist, axis=1, keepdims=True)
    ii = jax.lax.broadcasted_iota(jnp.int32, dist.shape, 1)
    idx = jnp.min(jnp.where(dist == m, ii, _NB), axis=1)
    oh = (ii == idx[:, None]).astype(jnp.float32)
    xd = _mm(oh, cb)
    xd_ref[...] = xd.reshape(T, B, CH)
    loss_ref[...] = prev_ref[...] + jnp.mean((xf - xd) ** 2).reshape(1, 1)
    p = jnp.sum(oh, axis=0) / (T * B)
    perp_ref[...] = jnp.exp(-jnp.sum(p * jnp.log(p + 1e-7))).reshape(1, 1)


def _full_spec(a):
    nd = a.ndim
    return pl.BlockSpec(a.shape, lambda i, _nd=nd: (0,) * _nd)


def _call(body, x, weights, bB, out_T, out_C, extra=()):
    B = x.shape[1]
    grid = (B // bB,)
    in_specs = [pl.BlockSpec((x.shape[0], bB, x.shape[2]), lambda i: (0, i, 0))]
    for e in extra:
        in_specs.append(pl.BlockSpec((e.shape[0], bB, e.shape[2]),
                                     lambda i: (0, i, 0)))
    in_specs += [_full_spec(w) for w in weights]
    return pl.pallas_call(
        body,
        grid=grid,
        in_specs=in_specs,
        out_specs=pl.BlockSpec((out_T, bB, out_C), lambda i: (0, i, 0)),
        out_shape=jax.ShapeDtypeStruct((out_T, B, out_C), jnp.float32),
        compiler_params=pltpu.CompilerParams(
            dimension_semantics=("arbitrary",)),
    )(x, *extra, *weights)


def _quantize(x, cb, prev_loss):
    T, B, CH = x.shape
    return pl.pallas_call(
        _quant_body,
        out_shape=(jax.ShapeDtypeStruct((T, B, CH), jnp.float32),
                   jax.ShapeDtypeStruct((1, 1), jnp.float32),
                   jax.ShapeDtypeStruct((1, 1), jnp.float32)),
    )(x, cb.T, cb, prev_loss)


def _cw(w):
    return jnp.transpose(w, (2, 1, 0))


def _bb(b):
    return b.reshape(1, -1)


def _stage_weights(dp, order):
    ws = []
    for j in order:
        rp = dp['res'][j]
        ws += [_cw(rp['w1']), _bb(rp['b1']), rp['w2'][:, :, 0].T, _bb(rp['b2'])]
    return ws


def _conv1d(x, w, b, stride=1, padding=1, dilation=1):
    y = jax.lax.conv_general_dilated(x, w, (stride,), [(padding, padding)],
                                     rhs_dilation=(dilation,),
                                     dimension_numbers=('NCH', 'OIH', 'NCH'))
    return y + b[None, :, None]


def _enc_resblock(x, p, dilation):
    h = jax.nn.relu(x)
    h = _conv1d(h, p['w1'], p['b1'], padding=dilation, dilation=dilation)
    h = jax.nn.relu(h)
    h = _conv1d(h, p['w2'], p['b2'], padding=0)
    return x + h


def _encoder(x, p):
    h = jax.nn.relu(_conv1d(x, p['in_w'], p['in_b']))
    for i in range(3):
        dp = p['down'][i]
        h = _conv1d(h, dp['cw'], dp['cb'], stride=2, padding=1)
        for j, dil in enumerate((1, 3, 9)):
            h = _enc_resblock(h, dp['res'][j], dil)
    return _conv1d(h, p['out_w'], p['out_b'])


def _decoder(h, p, bBs, out_w_full, out_b_full, other=None):
    h = _call(_in_conv_body, h, [_cw(p['in_w']), _bb(p['in_b'])], 64,
              h.shape[0], _C)
    for i in range(3):
        dp = p['up'][i]
        ws = _stage_weights(dp, (0, 1, 2)) + [_cw(dp['cw']), _bb(dp['cb'])]
        h = _call(_dec_stage_body, h, ws, bBs[i], h.shape[0] * 2, _C)
    ws = [_cw(p['mid_w']), _bb(p['mid_b']), out_w_full, out_b_full]
    if other is None:
        return _call(_dec_out_body, h, ws, 8, h.shape[0], _OUT)
    return _call(_dec_out_add_body, h, ws, 8, h.shape[0], _OUT, extra=(other,))


def kernel(x, enc_up, enc_low, cb_up, cb_low, dec_up, dec_low):
    upper = jnp.transpose(x[:, :, _UPIDX], (0, 2, 1)).astype(jnp.float32)
    lower = jnp.transpose(x[:, :, _LOIDX], (0, 2, 1)).astype(jnp.float32)
    eu = jnp.transpose(_encoder(upper, enc_up), (2, 0, 1))
    el = jnp.transpose(_encoder(lower, enc_low), (2, 0, 1))

    zero = jnp.zeros((1, 1), jnp.float32)
    uq, loss_u, _ = _quantize(eu, cb_up, zero)
    lq, loss, perp = _quantize(el, cb_low, loss_u)

    dec_bBs = (64, 32, 16)
    ow = jnp.zeros((3, _C, _OUT), jnp.float32).at[:, :, _UPIDX].set(
        _cw(dec_up['out_w']))
    ob = jnp.zeros((1, _OUT), jnp.float32).at[0, _UPIDX].set(dec_up['out_b'])
    du = _decoder(uq, dec_up, dec_bBs, ow, ob)
    ow2 = jnp.zeros((3, _C, _OUT), jnp.float32).at[:, :, _LOIDX].set(
        _cw(dec_low['out_w']))
    ob2 = jnp.zeros((1, _OUT), jnp.float32).at[0, _LOIDX].set(dec_low['out_b'])
    out = _decoder(lq, dec_low, dec_bBs, ow2, ob2, other=du)

    return (jnp.transpose(out, (1, 0, 2)),
            loss.reshape(()), perp.reshape(()))


# speedup vs baseline: 1.0271x; 1.0084x over previous
import numpy as np
import jax
import jax.numpy as jnp
from jax.experimental import pallas as pl
from jax.experimental.pallas import tpu as pltpu

_NB = 512
_CH = 256
_C = 512
_OUT = 263
_UP = 156

_perm = np.random.default_rng(42).permutation(_OUT)
_UPIDX = np.sort(_perm[:_UP])
_LOIDX = np.sort(_perm[_UP:])


def _mm(x, w):
    return jax.lax.dot_general(x, w, (((1,), (0,)), ((), ())),
                               preferred_element_type=jnp.float32)


def _conv3(h, w, b, dil):
    T, bB, C = h.shape
    N = w.shape[2]
    z = jnp.zeros((dil, bB, C), jnp.float32)
    hp = jnp.concatenate([z, h, z], axis=0)
    acc = _mm(hp[0:T].reshape(T * bB, C), w[0])
    acc = acc + _mm(hp[dil:dil + T].reshape(T * bB, C), w[1])
    acc = acc + _mm(hp[2 * dil:2 * dil + T].reshape(T * bB, C), w[2])
    return (acc + b).reshape(T, bB, N)


def _resblock(hx, w1, b1, w2, b2, dil):
    T, bB, C = hx.shape
    a = jnp.maximum(hx, 0.0)
    a = jnp.maximum(_conv3(a, w1, b1, dil), 0.0)
    h = _mm(a.reshape(T * bB, C), w2) + b2
    return hx + h.reshape(T, bB, C)


def _in_conv_body(x_ref, w_ref, b_ref, o_ref):
    o_ref[...] = jnp.maximum(_conv3(x_ref[...], w_ref[...], b_ref[...], 1), 0.0)


def _dec_stage_body(x_ref, *refs):
    o_ref = refs[-1]
    ws = [r[...] for r in refs[:-1]]
    h = x_ref[...]
    k = 0
    for dil in (9, 3, 1):
        h = _resblock(h, ws[k], ws[k + 1], ws[k + 2], ws[k + 3], dil)
        k += 4
    T, bB, C = h.shape
    h = jnp.broadcast_to(h[:, None], (T, 2, bB, C)).reshape(2 * T, bB, C)
    o_ref[...] = _conv3(h, ws[k], ws[k + 1], 1)


def _dec_out_body(x_ref, mw_ref, mb_ref, ow_ref, ob_ref, o_ref):
    h = jnp.maximum(_conv3(x_ref[...], mw_ref[...], mb_ref[...], 1), 0.0)
    o_ref[...] = _conv3(h, ow_ref[...], ob_ref[...], 1)


def _dec_out_add_body(x_ref, other_ref, mw_ref, mb_ref, ow_ref, ob_ref, o_ref):
    h = jnp.maximum(_conv3(x_ref[...], mw_ref[...], mb_ref[...], 1), 0.0)
    o_ref[...] = other_ref[...] + _conv3(h, ow_ref[...], ob_ref[...], 1)


def _quant_body(x_ref, cbt_ref, cb_ref, prev_ref, xd_ref, loss_ref, perp_ref):
    T, B, CH = x_ref.shape
    xf = x_ref[...].reshape(T * B, CH)
    cbt = cbt_ref[...]
    cb = cb_ref[...]
    mm = _mm(xf, cbt)
    xn = jnp.sum(xf * xf, axis=1, keepdims=True)
    cn = jnp.sum(cb * cb, axis=1)
    dist = xn - 2.0 * mm + cn[None, :]
    m = jnp.min(dist, axis=1, keepdims=True)
    ii = jax.lax.broadcasted_iota(jnp.int32, dist.shape, 1)
    idx = jnp.min(jnp.where(dist == m, ii, _NB), axis=1)
    oh = (ii == idx[:, None]).astype(jnp.float32)
    xd = _mm(oh, cb)
    xd_ref[...] = xd.reshape(T, B, CH)
    loss_ref[...] = prev_ref[...] + jnp.mean((xf - xd) ** 2).reshape(1, 1)
    p = jnp.sum(oh, axis=0) / (T * B)
    perp_ref[...] = jnp.exp(-jnp.sum(p * jnp.log(p + 1e-7))).reshape(1, 1)


def _full_spec(a):
    nd = a.ndim
    return pl.BlockSpec(a.shape, lambda i, _nd=nd: (0,) * _nd)


def _call(body, x, weights, bB, out_T, out_C, extra=()):
    B = x.shape[1]
    grid = (B // bB,)
    in_specs = [pl.BlockSpec((x.shape[0], bB, x.shape[2]), lambda i: (0, i, 0))]
    for e in extra:
        in_specs.append(pl.BlockSpec((e.shape[0], bB, e.shape[2]),
                                     lambda i: (0, i, 0)))
    in_specs += [_full_spec(w) for w in weights]
    return pl.pallas_call(
        body,
        grid=grid,
        in_specs=in_specs,
        out_specs=pl.BlockSpec((out_T, bB, out_C), lambda i: (0, i, 0)),
        out_shape=jax.ShapeDtypeStruct((out_T, B, out_C), jnp.float32),
        compiler_params=pltpu.CompilerParams(
            dimension_semantics=("arbitrary",)),
    )(x, *extra, *weights)


def _quantize(x, cb, prev_loss):
    T, B, CH = x.shape
    return pl.pallas_call(
        _quant_body,
        out_shape=(jax.ShapeDtypeStruct((T, B, CH), jnp.float32),
                   jax.ShapeDtypeStruct((1, 1), jnp.float32),
                   jax.ShapeDtypeStruct((1, 1), jnp.float32)),
    )(x, cb.T, cb, prev_loss)


def _cw(w):
    return jnp.transpose(w, (2, 1, 0))


def _bb(b):
    return b.reshape(1, -1)


def _stage_weights(dp, order):
    ws = []
    for j in order:
        rp = dp['res'][j]
        ws += [_cw(rp['w1']), _bb(rp['b1']), rp['w2'][:, :, 0].T, _bb(rp['b2'])]
    return ws


def _conv1d(x, w, b, stride=1, padding=1, dilation=1):
    y = jax.lax.conv_general_dilated(x, w, (stride,), [(padding, padding)],
                                     rhs_dilation=(dilation,),
                                     dimension_numbers=('NCH', 'OIH', 'NCH'))
    return y + b[None, :, None]


def _enc_resblock(x, p, dilation):
    h = jax.nn.relu(x)
    h = _conv1d(h, p['w1'], p['b1'], padding=dilation, dilation=dilation)
    h = jax.nn.relu(h)
    h = _conv1d(h, p['w2'], p['b2'], padding=0)
    return x + h


def _encoder(x, p):
    h = jax.nn.relu(_conv1d(x, p['in_w'], p['in_b']))
    for i in range(3):
        dp = p['down'][i]
        h = _conv1d(h, dp['cw'], dp['cb'], stride=2, padding=1)
        for j, dil in enumerate((1, 3, 9)):
            h = _enc_resblock(h, dp['res'][j], dil)
    return _conv1d(h, p['out_w'], p['out_b'])


def _decoder(h, p, bBs, out_w_full, out_b_full, other=None):
    h = _call(_in_conv_body, h, [_cw(p['in_w']), _bb(p['in_b'])], 128,
              h.shape[0], _C)
    for i in range(3):
        dp = p['up'][i]
        ws = _stage_weights(dp, (0, 1, 2)) + [_cw(dp['cw']), _bb(dp['cb'])]
        h = _call(_dec_stage_body, h, ws, bBs[i], h.shape[0] * 2, _C)
    ws = [_cw(p['mid_w']), _bb(p['mid_b']), out_w_full, out_b_full]
    if other is None:
        return _call(_dec_out_body, h, ws, 16, h.shape[0], _OUT)
    return _call(_dec_out_add_body, h, ws, 16, h.shape[0], _OUT, extra=(other,))


def kernel(x, enc_up, enc_low, cb_up, cb_low, dec_up, dec_low):
    upper = jnp.transpose(x[:, :, _UPIDX], (0, 2, 1)).astype(jnp.float32)
    lower = jnp.transpose(x[:, :, _LOIDX], (0, 2, 1)).astype(jnp.float32)
    eu = jnp.transpose(_encoder(upper, enc_up), (2, 0, 1))
    el = jnp.transpose(_encoder(lower, enc_low), (2, 0, 1))

    zero = jnp.zeros((1, 1), jnp.float32)
    uq, loss_u, _ = _quantize(eu, cb_up, zero)
    lq, loss, perp = _quantize(el, cb_low, loss_u)

    dec_bBs = (128, 64, 32)
    ow = jnp.zeros((3, _C, _OUT), jnp.float32).at[:, :, _UPIDX].set(
        _cw(dec_up['out_w']))
    ob = jnp.zeros((1, _OUT), jnp.float32).at[0, _UPIDX].set(dec_up['out_b'])
    du = _decoder(uq, dec_up, dec_bBs, ow, ob)
    ow2 = jnp.zeros((3, _C, _OUT), jnp.float32).at[:, :, _LOIDX].set(
        _cw(dec_low['out_w']))
    ob2 = jnp.zeros((1, _OUT), jnp.float32).at[0, _LOIDX].set(dec_low['out_b'])
    out = _decoder(lq, dec_low, dec_bBs, ow2, ob2, other=du)

    return (jnp.transpose(out, (1, 0, 2)),
            loss.reshape(()), perp.reshape(()))
